# baseline (speedup 1.0000x reference)
import jax
import jax.numpy as jnp
from jax import lax
from jax.experimental import pallas as pl
from jax.experimental.pallas import tpu as pltpu
from jax.experimental.pallas import tpu_sc as plsc

NC = 2
NS = 16
NW = NC * NS
BSZ = 1024
NBUF = 3


def _make_sc_agg(n, per_tile, e, with_cnt):
  n_full, tail = divmod(per_tile, BSZ)
  total = n_full + (1 if tail else 0)
  epi = (total % NBUF) + NBUF
  n_loop = total - epi
  assert n_loop >= 0 and n_loop % NBUF == 0 and total >= 2
  assert tail % 8 == 0
  rpt = -(-n // NS)
  rpt += -rpt % 8
  last_rows = n - (NS - 1) * rpt
  assert 0 < last_rows <= rpt
  mesh = plsc.VectorSubcoreMesh(core_axis_name="c", subcore_axis_name="s")

  out_type = [jax.ShapeDtypeStruct((NC, n, 16), jnp.float32)]
  scratch = (
      [pltpu.VMEM((per_tile,), jnp.int32)] * 2
      + [pltpu.VMEM((BSZ, 16), jnp.float32)] * NBUF
      + [pltpu.VMEM((rpt, 16), jnp.float32)]
      + [pltpu.VMEM_SHARED((n, 16), jnp.float32)]
      + [pltpu.SemaphoreType.DMA] * (2 * NBUF)
  )
  if with_cnt:
    out_type.append(jax.ShapeDtypeStruct((NC, n, 16), jnp.float32))
    scratch += [
        pltpu.VMEM((BSZ, 16), jnp.float32),
        pltpu.VMEM_SHARED((n, 16), jnp.float32),
        pltpu.SemaphoreType.DMA,
    ]

  def body(table, ei, *rest):
    if with_cnt:
      out_f, out_c = rest[0], rest[1]
      rest = rest[2:]
      ones_v, cnt_s, csem = rest[-3:]
    else:
      out_f = rest[0]
      rest = rest[1:]
      out_c = ones_v = cnt_s = csem = None
    src_v, dst_v = rest[0], rest[1]
    rows = rest[2:2 + NBUF]
    stripe_v = rest[2 + NBUF]
    acc_s = rest[3 + NBUF]
    gsem = rest[4 + NBUF:4 + 2 * NBUF]
    ssem = rest[4 + 2 * NBUF:4 + 3 * NBUF]
    c = lax.axis_index("c")
    s = lax.axis_index("s")
    w = s * NC + c

    assert rpt % 8 == 0
    def zrow(i, _):
      for j in range(8):
        stripe_v[i * 8 + j, :] = jnp.zeros((16,), jnp.float32)
      return _
    lax.fori_loop(0, rpt // 8, zrow, None)

    def zero_acc(acc):
      @pl.when(s < NS - 1)
      def _():
        pltpu.sync_copy(stripe_v, acc.at[pl.ds(s * rpt, rpt)])

      @pl.when(s == NS - 1)
      def _():
        pltpu.sync_copy(stripe_v.at[pl.ds(0, last_rows)],
                        acc.at[pl.ds((NS - 1) * rpt, last_rows)])

    zero_acc(acc_s)
    if with_cnt:
      zero_acc(cnt_s)

      def orow(i, _):
        for j in range(8):
          ones_v[i * 8 + j, :] = jnp.ones((16,), jnp.float32)
        return _
      lax.fori_loop(0, BSZ // 8, orow, None)

    base = w * per_tile
    pltpu.sync_copy(ei.at[pl.ds(base, per_tile)], src_v)
    pltpu.sync_copy(ei.at[pl.ds(e + base, per_tile)], dst_v)
    plsc.subcore_barrier()

    def blk_sz(bi):
      return tail if (tail and bi == total - 1) else BSZ

    def rbuf(b, sz):
      return rows[b] if sz == BSZ else rows[b].at[pl.ds(0, sz)]

    def gath(bi, b, sz=BSZ):
      return pltpu.make_async_copy(
          table.at[src_v.at[pl.ds(bi * BSZ, sz)]], rbuf(b, sz), gsem[b])

    def scat(bi, b, sz=BSZ):
      return pltpu.make_async_copy(
          rbuf(b, sz), acc_s.at[dst_v.at[pl.ds(bi * BSZ, sz)]], ssem[b])

    def cscat(bi, sz=BSZ):
      ones = ones_v if sz == BSZ else ones_v.at[pl.ds(0, sz)]
      return pltpu.make_async_copy(
          ones, cnt_s.at[dst_v.at[pl.ds(bi * BSZ, sz)]], csem)

    gath(0, 0).start()
    gath(1, 1).start()

    def maybe(bi, cond, fn):
      if isinstance(bi, int):
        if cond:
          fn()
      else:
        pl.when(cond)(fn)

    def step(bi, b, nb, sz, ahead_sz):
      maybe(bi, bi >= NBUF - 2, lambda: scat(bi + 2 - NBUF, nb).wait())
      if ahead_sz:
        gath(bi + 2, nb, ahead_sz).start()
      gath(bi, b, sz).wait()
      scat(bi, b, sz).start(add=True)
      if with_cnt:
        cscat(bi, sz).start(add=True)
        maybe(bi, bi >= NBUF, lambda: cscat(bi - NBUF).wait())

    def quad(qi, _):
      for b in range(NBUF):
        bi = qi * NBUF + b
        step(bi, b, (b + 2) % NBUF, BSZ, ahead_sz=BSZ)
      return _
    lax.fori_loop(0, n_loop // NBUF, quad, None)

    for bi in range(n_loop, total):
      b = bi % NBUF
      ahead = blk_sz(bi + 2) if bi + 2 < total else None
      step(bi, b, (b + 2) % NBUF, blk_sz(bi), ahead_sz=ahead)

    for bi in range(total - (NBUF - 2), total):
      scat(bi, bi % NBUF, blk_sz(bi)).wait()
    if with_cnt:
      for bi in range(max(0, total - NBUF), total):
        cscat(bi, blk_sz(bi)).wait()
    plsc.subcore_barrier()

    def writeback(acc, out):
      @pl.when(s < NS - 1)
      def _():
        sl = pl.ds(s * rpt, rpt)
        pltpu.sync_copy(acc.at[sl], out.at[c, sl])

      @pl.when(s == NS - 1)
      def _():
        sl = pl.ds((NS - 1) * rpt, last_rows)
        pltpu.sync_copy(acc.at[sl], out.at[c, sl])

    writeback(acc_s, out_f)
    if with_cnt:
      writeback(cnt_s, out_c)

  return pl.kernel(
      body, out_type, mesh=mesh, scratch_types=scratch,
      compiler_params=pltpu.CompilerParams(use_tc_tiling_on_sc=False))


def _make_sc_cnt(n, per_tile, e):
  n_full, tail = divmod(per_tile, BSZ)
  total = n_full + (1 if tail else 0)
  rpt = -(-n // NS)
  rpt += -rpt % 8
  last_rows = n - (NS - 1) * rpt
  assert 0 < last_rows <= rpt and rpt % 8 == 0 and tail % 8 == 0
  mesh = plsc.VectorSubcoreMesh(core_axis_name="c", subcore_axis_name="s")

  out_type = [jax.ShapeDtypeStruct((NC, n, 16), jnp.float32)]
  scratch = [
      pltpu.VMEM((per_tile,), jnp.int32),
      pltpu.VMEM((BSZ, 16), jnp.float32),
      pltpu.VMEM((rpt, 16), jnp.float32),
      pltpu.VMEM_SHARED((n, 16), jnp.float32),
      pltpu.SemaphoreType.DMA,
  ]

  def body(ei, out_c, dst_v, ones_v, stripe_v, cnt_s, csem):
    c = lax.axis_index("c")
    s = lax.axis_index("s")
    w = s * NC + c

    def zrow(i, _):
      for j in range(8):
        stripe_v[i * 8 + j, :] = jnp.zeros((16,), jnp.float32)
      return _
    lax.fori_loop(0, rpt // 8, zrow, None)

    def orow(i, _):
      for j in range(8):
        ones_v[i * 8 + j, :] = jnp.ones((16,), jnp.float32)
      return _
    lax.fori_loop(0, BSZ // 8, orow, None)

    @pl.when(s < NS - 1)
    def _():
      pltpu.sync_copy(stripe_v, cnt_s.at[pl.ds(s * rpt, rpt)])

    @pl.when(s == NS - 1)
    def _():
      pltpu.sync_copy(stripe_v.at[pl.ds(0, last_rows)],
                      cnt_s.at[pl.ds((NS - 1) * rpt, last_rows)])

    pltpu.sync_copy(ei.at[pl.ds(e + w * per_tile, per_tile)], dst_v)
    plsc.subcore_barrier()

    def blk_sz(bi):
      return tail if (tail and bi == total - 1) else BSZ

    def cscat(bi, sz=BSZ):
      ones = ones_v if sz == BSZ else ones_v.at[pl.ds(0, sz)]
      return pltpu.make_async_copy(
          ones, cnt_s.at[dst_v.at[pl.ds(bi * BSZ, sz)]], csem)

    def step(bi, _):
      cscat(bi).start(add=True)

      @pl.when(bi >= NBUF)
      def _():
        cscat(bi - NBUF).wait()
      return _
    lax.fori_loop(0, total - 1, step, None)
    cscat(total - 1, blk_sz(total - 1)).start(add=True)
    for bi in range(max(0, total - 1 - NBUF), total - 1):
      cscat(bi).wait()
    cscat(total - 1, blk_sz(total - 1)).wait()
    plsc.subcore_barrier()

    @pl.when(s < NS - 1)
    def _():
      sl = pl.ds(s * rpt, rpt)
      pltpu.sync_copy(cnt_s.at[sl], out_c.at[c, sl])

    @pl.when(s == NS - 1)
    def _():
      sl = pl.ds((NS - 1) * rpt, last_rows)
      pltpu.sync_copy(cnt_s.at[sl], out_c.at[c, sl])

  return pl.kernel(
      body, out_type, mesh=mesh, scratch_types=scratch,
      compiler_params=pltpu.CompilerParams(use_tc_tiling_on_sc=False))


def _tc1_body(x3_ref, wl_ref, wr_ref, pl_ref, pr_ref):
  wl = wl_ref[...]
  wr = wr_ref[...]
  for j in range(8):
    xj = x3_ref[:, j, :]
    pl_ref[:, j * 16:(j + 1) * 16] = jnp.dot(
        xj, wl, preferred_element_type=jnp.float32)
    pr_ref[:, j * 16:(j + 1) * 16] = jnp.dot(
        xj, wr, preferred_element_type=jnp.float32)


def _tc2_body(accf_ref, accc_ref, xr_ref, b1_ref, wl_ref, wr_ref,
              hp_ref, hr_ref):
  f = accf_ref[0] + accf_ref[1]
  cnt = accc_ref[0] + accc_ref[1]
  mean = f / jnp.maximum(cnt, 1.0)
  h = jnp.maximum(mean + b1_ref[...] + xr_ref[...], 0.0)
  hp_ref[...] = jnp.dot(h, wl_ref[...], preferred_element_type=jnp.float32)
  hr_ref[...] = jnp.dot(h, wr_ref[...], preferred_element_type=jnp.float32)


def _tc3_body(accf_ref, accc_ref, hr_ref, b2_ref, onesbd_ref, out_ref):
  f = accf_ref[0] + accf_ref[1]
  cnt = accc_ref[0] + accc_ref[1]
  z = f / jnp.maximum(cnt, 1.0) + b2_ref[...] + hr_ref[...]
  off = jax.lax.broadcasted_iota(jnp.int32, z.shape, 1) % 16
  m = z
  for sft in (1, 2, 4, 8):
    right = pltpu.roll(m, 128 - sft, 1)
    left = pltpu.roll(m, sft, 1)
    m = jnp.where(off + sft < 16, jnp.maximum(m, right), m)
    m = jnp.where(off >= sft, jnp.maximum(m, left), m)
  zs = z - m
  e = jnp.exp(zs)
  s_all = jnp.dot(e, onesbd_ref[...], preferred_element_type=jnp.float32)
  out_ref[...] = zs - jnp.log(s_all)


def _block_diag8(w):
  return jnp.einsum("jJ,kf->jkJf", jnp.eye(8, dtype=w.dtype), w).reshape(
      8 * w.shape[0], 8 * w.shape[1])


def kernel(x, edge_index, W1l, b1, W1r, W2l, b2, W2r):
  n, f_in = x.shape
  e = edge_index.shape[1]
  h = W1l.shape[1]
  assert h == 16 and W2l.shape[1] == 16 and n % 8 == 0

  per_tile, rem = divmod(e, NW)
  assert rem == 0 and per_tile % 8 == 0

  npk = n // 8
  blk = npk
  grid = 1

  ei = edge_index.astype(jnp.int32).reshape(2 * e)
  x3 = x.reshape(npk, 8, f_in)
  Wd2l = _block_diag8(W2l)
  Wd2r = _block_diag8(W2r)
  b1t = jnp.tile(b1, 8).reshape(1, 8 * h)
  b2t = jnp.tile(b2, 8).reshape(1, 8 * h)

  pk_spec = pl.BlockSpec((blk, 128), lambda i: (i, 0))
  acc_spec = pl.BlockSpec((NC, blk, 128), lambda i: (0, i, 0))
  wd_spec = pl.BlockSpec((128, 128), lambda i: (0, 0))
  b_spec = pl.BlockSpec((1, 128), lambda i: (0, 0))
  x3_spec = pl.BlockSpec((blk, 8, f_in), lambda i: (i, 0, 0))
  w1_spec = pl.BlockSpec((f_in, h), lambda i: (0, 0))
  pk_shape = jax.ShapeDtypeStruct((npk, 128), jnp.float32)

  cntk = _make_sc_cnt(n, per_tile, e)
  (acc_cnt,) = cntk(ei)

  xp1, xr1 = pl.pallas_call(
      _tc1_body, grid=(grid,), in_specs=[x3_spec, w1_spec, w1_spec],
      out_specs=[pk_spec] * 2, out_shape=[pk_shape] * 2)(x3, W1l, W1r)

  agg1 = _make_sc_agg(n, per_tile, e, with_cnt=False)
  (acc1,) = agg1(xp1.reshape(n, h), ei)
  acc1 = acc1.reshape(NC, npk, 128)
  acc_cnt = acc_cnt.reshape(NC, npk, 128)

  hp2, hr2 = pl.pallas_call(
      _tc2_body, grid=(grid,),
      in_specs=[acc_spec, acc_spec, pk_spec, b_spec, wd_spec, wd_spec],
      out_specs=[pk_spec] * 2, out_shape=[pk_shape] * 2,
  )(acc1, acc_cnt, xr1, b1t, Wd2l, Wd2r)

  agg2 = _make_sc_agg(n, per_tile, e, with_cnt=False)
  (acc2,) = agg2(hp2.reshape(n, h), ei)
  acc2 = acc2.reshape(NC, npk, 128)

  onesbd = _block_diag8(jnp.ones((h, h), jnp.float32))
  out_pk = pl.pallas_call(
      _tc3_body, grid=(grid,),
      in_specs=[acc_spec, acc_spec, pk_spec, b_spec, wd_spec],
      out_specs=pk_spec, out_shape=pk_shape,
  )(acc2, acc_cnt, hr2, b2t, onesbd)

  return out_pk.reshape(n, h)


# speedup vs baseline: 1.0675x; 1.0675x over previous
import jax
import jax.numpy as jnp
from jax import lax
from jax.experimental import pallas as pl
from jax.experimental.pallas import tpu as pltpu
from jax.experimental.pallas import tpu_sc as plsc

NC = 2
NS = 16
NW = NC * NS
BSZ = 1024
NBUF = 3


def _make_sc_agg(n, per_tile, e, with_cnt):
  n_full, tail = divmod(per_tile, BSZ)
  total = n_full + (1 if tail else 0)
  epi = (total % NBUF) + NBUF
  n_loop = total - epi
  assert n_loop >= 0 and n_loop % NBUF == 0 and total >= 2
  assert tail % 8 == 0
  rpt = -(-n // NS)
  rpt += -rpt % 8
  last_rows = n - (NS - 1) * rpt
  assert 0 < last_rows <= rpt
  mesh = plsc.VectorSubcoreMesh(core_axis_name="c", subcore_axis_name="s")

  out_type = [jax.ShapeDtypeStruct((NC, n, 16), jnp.float32)]
  scratch = (
      [pltpu.VMEM((per_tile,), jnp.int32)] * 2
      + [pltpu.VMEM((BSZ, 16), jnp.float32)] * NBUF
      + [pltpu.VMEM((rpt, 16), jnp.float32)]
      + [pltpu.VMEM_SHARED((n, 16), jnp.float32)]
      + [pltpu.SemaphoreType.DMA] * (2 * NBUF)
  )
  if with_cnt:
    out_type.append(jax.ShapeDtypeStruct((NC, n, 16), jnp.float32))
    scratch += [
        pltpu.VMEM((BSZ, 16), jnp.float32),
        pltpu.VMEM_SHARED((n, 16), jnp.float32),
        pltpu.SemaphoreType.DMA,
    ]

  def body(table, ei, *rest):
    if with_cnt:
      out_f, out_c = rest[0], rest[1]
      rest = rest[2:]
      ones_v, cnt_s, csem = rest[-3:]
    else:
      out_f = rest[0]
      rest = rest[1:]
      out_c = ones_v = cnt_s = csem = None
    src_v, dst_v = rest[0], rest[1]
    rows = rest[2:2 + NBUF]
    stripe_v = rest[2 + NBUF]
    acc_s = rest[3 + NBUF]
    gsem = rest[4 + NBUF:4 + 2 * NBUF]
    ssem = rest[4 + 2 * NBUF:4 + 3 * NBUF]
    c = lax.axis_index("c")
    s = lax.axis_index("s")
    w = s * NC + c

    assert rpt % 8 == 0
    def zrow(i, _):
      for j in range(8):
        stripe_v[i * 8 + j, :] = jnp.zeros((16,), jnp.float32)
      return _
    lax.fori_loop(0, rpt // 8, zrow, None)

    def zero_acc(acc):
      @pl.when(s < NS - 1)
      def _():
        pltpu.sync_copy(stripe_v, acc.at[pl.ds(s * rpt, rpt)])

      @pl.when(s == NS - 1)
      def _():
        pltpu.sync_copy(stripe_v.at[pl.ds(0, last_rows)],
                        acc.at[pl.ds((NS - 1) * rpt, last_rows)])

    zero_acc(acc_s)
    if with_cnt:
      zero_acc(cnt_s)

      def orow(i, _):
        for j in range(8):
          ones_v[i * 8 + j, :] = jnp.ones((16,), jnp.float32)
        return _
      lax.fori_loop(0, BSZ // 8, orow, None)

    base = w * per_tile
    pltpu.sync_copy(ei.at[pl.ds(base, per_tile)], src_v)
    pltpu.sync_copy(ei.at[pl.ds(e + base, per_tile)], dst_v)
    plsc.subcore_barrier()

    def blk_sz(bi):
      return tail if (tail and bi == total - 1) else BSZ

    def rbuf(b, sz):
      return rows[b] if sz == BSZ else rows[b].at[pl.ds(0, sz)]

    def gath(bi, b, sz=BSZ):
      return pltpu.make_async_copy(
          table.at[src_v.at[pl.ds(bi * BSZ, sz)]], rbuf(b, sz), gsem[b])

    def scat(bi, b, sz=BSZ):
      return pltpu.make_async_copy(
          rbuf(b, sz), acc_s.at[dst_v.at[pl.ds(bi * BSZ, sz)]], ssem[b])

    def cscat(bi, sz=BSZ):
      ones = ones_v if sz == BSZ else ones_v.at[pl.ds(0, sz)]
      return pltpu.make_async_copy(
          ones, cnt_s.at[dst_v.at[pl.ds(bi * BSZ, sz)]], csem)

    gath(0, 0).start()
    gath(1, 1).start()

    def maybe(bi, cond, fn):
      if isinstance(bi, int):
        if cond:
          fn()
      else:
        pl.when(cond)(fn)

    def step(bi, b, nb, sz, ahead_sz):
      maybe(bi, bi >= NBUF - 2, lambda: scat(bi + 2 - NBUF, nb).wait())
      if ahead_sz:
        gath(bi + 2, nb, ahead_sz).start()
      gath(bi, b, sz).wait()
      scat(bi, b, sz).start(add=True)
      if with_cnt:
        cscat(bi, sz).start(add=True)
        maybe(bi, bi >= NBUF, lambda: cscat(bi - NBUF).wait())

    def quad(qi, _):
      for b in range(NBUF):
        bi = qi * NBUF + b
        step(bi, b, (b + 2) % NBUF, BSZ, ahead_sz=BSZ)
      return _
    lax.fori_loop(0, n_loop // NBUF, quad, None)

    for bi in range(n_loop, total):
      b = bi % NBUF
      ahead = blk_sz(bi + 2) if bi + 2 < total else None
      step(bi, b, (b + 2) % NBUF, blk_sz(bi), ahead_sz=ahead)

    for bi in range(total - (NBUF - 2), total):
      scat(bi, bi % NBUF, blk_sz(bi)).wait()
    if with_cnt:
      for bi in range(max(0, total - NBUF), total):
        cscat(bi, blk_sz(bi)).wait()
    plsc.subcore_barrier()

    def writeback(acc, out):
      @pl.when(s < NS - 1)
      def _():
        sl = pl.ds(s * rpt, rpt)
        pltpu.sync_copy(acc.at[sl], out.at[c, sl])

      @pl.when(s == NS - 1)
      def _():
        sl = pl.ds((NS - 1) * rpt, last_rows)
        pltpu.sync_copy(acc.at[sl], out.at[c, sl])

    writeback(acc_s, out_f)
    if with_cnt:
      writeback(cnt_s, out_c)

  return pl.kernel(
      body, out_type, mesh=mesh, scratch_types=scratch,
      compiler_params=pltpu.CompilerParams(use_tc_tiling_on_sc=False))


def _tc1_body(x3_ref, wl_ref, wr_ref, pl_ref, pr_ref):
  wl = wl_ref[...]
  wr = wr_ref[...]
  for j in range(8):
    xj = x3_ref[:, j, :]
    pl_ref[:, j * 16:(j + 1) * 16] = jnp.dot(
        xj, wl, preferred_element_type=jnp.float32)
    pr_ref[:, j * 16:(j + 1) * 16] = jnp.dot(
        xj, wr, preferred_element_type=jnp.float32)


def _tc2_body(accf_ref, accc_ref, xr_ref, b1_ref, wl_ref, wr_ref,
              hp_ref, hr_ref):
  f = accf_ref[0] + accf_ref[1]
  cnt = accc_ref[0] + accc_ref[1]
  mean = f / jnp.maximum(cnt, 1.0)
  h = jnp.maximum(mean + b1_ref[...] + xr_ref[...], 0.0)
  hp_ref[...] = jnp.dot(h, wl_ref[...], preferred_element_type=jnp.float32)
  hr_ref[...] = jnp.dot(h, wr_ref[...], preferred_element_type=jnp.float32)


def _tc3_body(accf_ref, accc_ref, hr_ref, b2_ref, onesbd_ref, out_ref):
  f = accf_ref[0] + accf_ref[1]
  cnt = accc_ref[0] + accc_ref[1]
  z = f / jnp.maximum(cnt, 1.0) + b2_ref[...] + hr_ref[...]
  off = jax.lax.broadcasted_iota(jnp.int32, z.shape, 1) % 16
  m = z
  for sft in (1, 2, 4, 8):
    right = pltpu.roll(m, 128 - sft, 1)
    left = pltpu.roll(m, sft, 1)
    m = jnp.where(off + sft < 16, jnp.maximum(m, right), m)
    m = jnp.where(off >= sft, jnp.maximum(m, left), m)
  zs = z - m
  e = jnp.exp(zs)
  s_all = jnp.dot(e, onesbd_ref[...], preferred_element_type=jnp.float32)
  out_ref[...] = zs - jnp.log(s_all)


def _block_diag8(w):
  return jnp.einsum("jJ,kf->jkJf", jnp.eye(8, dtype=w.dtype), w).reshape(
      8 * w.shape[0], 8 * w.shape[1])


def kernel(x, edge_index, W1l, b1, W1r, W2l, b2, W2r):
  n, f_in = x.shape
  e = edge_index.shape[1]
  h = W1l.shape[1]
  assert h == 16 and W2l.shape[1] == 16 and n % 8 == 0

  per_tile, rem = divmod(e, NW)
  assert rem == 0 and per_tile % 8 == 0

  npk = n // 8
  blk = npk
  grid = 1

  ei = edge_index.astype(jnp.int32).reshape(2 * e)
  x3 = x.reshape(npk, 8, f_in)
  Wd2l = _block_diag8(W2l)
  Wd2r = _block_diag8(W2r)
  b1t = jnp.tile(b1, 8).reshape(1, 8 * h)
  b2t = jnp.tile(b2, 8).reshape(1, 8 * h)

  pk_spec = pl.BlockSpec((blk, 128), lambda i: (i, 0))
  acc_spec = pl.BlockSpec((NC, blk, 128), lambda i: (0, i, 0))
  wd_spec = pl.BlockSpec((128, 128), lambda i: (0, 0))
  b_spec = pl.BlockSpec((1, 128), lambda i: (0, 0))
  x3_spec = pl.BlockSpec((blk, 8, f_in), lambda i: (i, 0, 0))
  w1_spec = pl.BlockSpec((f_in, h), lambda i: (0, 0))
  pk_shape = jax.ShapeDtypeStruct((npk, 128), jnp.float32)

  xp1, xr1 = pl.pallas_call(
      _tc1_body, grid=(grid,), in_specs=[x3_spec, w1_spec, w1_spec],
      out_specs=[pk_spec] * 2, out_shape=[pk_shape] * 2)(x3, W1l, W1r)

  agg1 = _make_sc_agg(n, per_tile, e, with_cnt=True)
  acc1, acc_cnt = agg1(xp1.reshape(n, h), ei)
  acc1 = acc1.reshape(NC, npk, 128)
  acc_cnt = acc_cnt.reshape(NC, npk, 128)

  hp2, hr2 = pl.pallas_call(
      _tc2_body, grid=(grid,),
      in_specs=[acc_spec, acc_spec, pk_spec, b_spec, wd_spec, wd_spec],
      out_specs=[pk_spec] * 2, out_shape=[pk_shape] * 2,
  )(acc1, acc_cnt, xr1, b1t, Wd2l, Wd2r)

  agg2 = _make_sc_agg(n, per_tile, e, with_cnt=False)
  (acc2,) = agg2(hp2.reshape(n, h), ei)
  acc2 = acc2.reshape(NC, npk, 128)

  onesbd = _block_diag8(jnp.ones((h, h), jnp.float32))
  out_pk = pl.pallas_call(
      _tc3_body, grid=(grid,),
      in_specs=[acc_spec, acc_spec, pk_spec, b_spec, wd_spec],
      out_specs=pk_spec, out_shape=pk_shape,
  )(acc2, acc_cnt, hr2, b2t, onesbd)

  return out_pk.reshape(n, h)
